# baseline (speedup 1.0000x reference)
import functools

import jax
import jax.numpy as jnp
from jax import lax
from jax.experimental import pallas as pl
from jax.experimental.pallas import tpu as pltpu
from jax.experimental.pallas import tpu_sc as plsc

_B = 16384 * 26
_B0 = 1000000
_D = 32
_C = 128
_NC, _NS = 2, 16
_NW = _NC * _NS
_PER_W = _B // _NW
_NCHUNK = _PER_W // _C


_PACK = 128 // _D
_RBO = 4096
_M = _PACK * _RBO
_NBLK = _B0 // _M
_MAIN = _NBLK * _M
_TAIL = _B0 - _MAIN
_TB = 64
_P = _NBLK * _RBO + _TAIL
_B0V = _P * _PACK


def _compact_block(i0, i1, i2, i3, o_ref):
    o_ref[...] = jnp.concatenate(
        [i0[...], i1[...], i2[...], i3[...]], axis=0
    ).T


def _tail_block(i_ref, full_ref, o_ref):
    del full_ref
    o_ref[...] = jnp.concatenate([i_ref[...]] * _PACK, axis=1)


def _compact_table(wt, weight):
    in_specs = [
        pl.BlockSpec(
            (_D, _RBO), functools.partial(lambda k, i: (0, 4 * i + k), k)
        )
        for k in range(_PACK)
    ]
    main = pl.pallas_call(
        _compact_block,
        grid=(_NBLK,),
        in_specs=in_specs,
        out_specs=pl.BlockSpec((_RBO, _PACK * _D), lambda i: (i, 0)),
        out_shape=jax.ShapeDtypeStruct((_P, _PACK * _D), jnp.float32),
        compiler_params=pltpu.CompilerParams(
            dimension_semantics=("parallel",)
        ),
    )(wt, wt, wt, wt)
    tail = jax.lax.slice(weight, (_MAIN, 0), (_B0, _D))
    return pl.pallas_call(
        _tail_block,
        grid=(_TAIL // _TB,),
        in_specs=[
            pl.BlockSpec((_TB, _D), lambda j: (j, 0)),
            pl.BlockSpec(memory_space=pl.ANY),
        ],
        out_specs=pl.BlockSpec((_TB, _PACK * _D), lambda j: (_NBLK * _RBO // _TB + j, 0)),
        out_shape=jax.ShapeDtypeStruct((_P, _PACK * _D), jnp.float32),
        input_output_aliases={1: 0},
    )(tail, main)


def kernel(x, weight):
    weight = _compact_table(weight.T, weight).reshape(_B0V, _D)
    idx = x.T.reshape(_B)
    t = idx % _M
    idx = jnp.where(
        idx < _MAIN,
        (idx - t) + _PACK * (t % _RBO) + t // _RBO,
        _PACK * idx - (_PACK - 1) * _MAIN,
    )
    mesh = plsc.VectorSubcoreMesh(core_axis_name="c", subcore_axis_name="s")

    @functools.partial(
        pl.kernel,
        mesh=mesh,
        out_type=jax.ShapeDtypeStruct((_B * _D,), jnp.float32),
        compiler_params=pltpu.CompilerParams(
            use_tc_tiling_on_sc=False, needs_layout_passes=False
        ),
        scratch_types=[
            pltpu.VMEM((_PER_W,), jnp.int32),
            pltpu.VMEM((_C, _D), jnp.float32),
            pltpu.VMEM((_C, _D), jnp.float32),
            pltpu.VMEM((_C * _D,), jnp.float32),
            pltpu.VMEM((_C * _D,), jnp.float32),
            pltpu.SemaphoreType.DMA,
            pltpu.SemaphoreType.DMA,
            pltpu.SemaphoreType.DMA,
            pltpu.SemaphoreType.DMA,
        ],
    )
    def gather_kernel(
        w_hbm, i_hbm, o_hbm, idx_v, rows0, rows1, tr0, tr1, sem0, sem1, ss0, ss1
    ):
        wid = lax.axis_index("s") * _NC + lax.axis_index("c")
        base = wid * _PER_W
        iota = lax.broadcasted_iota(jnp.int32, (16,), 0)

        pltpu.sync_copy(i_hbm.at[pl.ds(base, _PER_W)], idx_v)

        def start(j, rows, sem):
            pltpu.async_copy(w_hbm.at[idx_v.at[pl.ds(j * _C, _C)]], rows, sem)

        def wait(rows, sem):
            pltpu.make_async_copy(w_hbm.at[pl.ds(0, _C)], rows, sem).wait()

        def transpose(rows, trans):
            @pl.loop(0, _D)
            def _(d):
                col = jnp.zeros((16,), jnp.int32) + d
                for g in range(8):
                    vals = plsc.load_gather(rows, [iota + 16 * g, col])
                    trans[pl.ds(d * _C + 16 * g, 16)] = vals

        def store(j, trans, ssem):
            w_id = wid * _NCHUNK + j
            c = w_id // _C
            off = c * 524288 + (w_id - c * _C) * 1024
            for tr in range(4):
                pltpu.async_copy(
                    trans.at[pl.ds(1024 * tr, 1024)],
                    o_hbm.at[pl.ds(off + tr * 131072, 1024)],
                    ssem,
                )

        def wait_store(trans, ssem):
            for tr in range(4):
                pltpu.make_async_copy(
                    o_hbm.at[pl.ds(0, 1024)], trans.at[pl.ds(0, 1024)], ssem
                ).wait()

        start(0, rows0, sem0)
        start(1, rows1, sem1)

        wait(rows0, sem0)
        transpose(rows0, tr0)
        start(2, rows0, sem0)
        store(0, tr0, ss0)
        wait(rows1, sem1)
        transpose(rows1, tr1)
        start(3, rows1, sem1)
        store(1, tr1, ss1)

        @pl.loop(2, _NCHUNK - 2, step=2)
        def _(j):
            wait(rows0, sem0)
            wait_store(tr0, ss0)
            transpose(rows0, tr0)
            start(j + 2, rows0, sem0)
            store(j, tr0, ss0)
            wait(rows1, sem1)
            wait_store(tr1, ss1)
            transpose(rows1, tr1)
            start(j + 3, rows1, sem1)
            store(j + 1, tr1, ss1)

        wait(rows0, sem0)
        wait_store(tr0, ss0)
        transpose(rows0, tr0)
        store(_NCHUNK - 2, tr0, ss0)
        wait(rows1, sem1)
        wait_store(tr1, ss1)
        transpose(rows1, tr1)
        store(_NCHUNK - 1, tr1, ss1)
        wait_store(tr0, ss0)
        wait_store(tr1, ss1)

    out = gather_kernel(weight, idx)
    out5 = out.reshape(26, 4, 128, 8, 128)
    return out5.transpose(2, 4, 0, 1, 3).reshape(16384, 26, 32)


# speedup vs baseline: 1.1547x; 1.1547x over previous
import functools

import jax
import jax.numpy as jnp
from jax import lax
from jax.experimental import pallas as pl
from jax.experimental.pallas import tpu as pltpu
from jax.experimental.pallas import tpu_sc as plsc

_B = 16384 * 26
_B0 = 1000000
_D = 32
_C = 128
_NC, _NS = 2, 16
_NW = _NC * _NS
_PER_W = _B // _NW
_NCHUNK = _PER_W // _C


_PACK = 128 // _D
_RBO = 4096
_M = _PACK * _RBO
_NBLK = _B0 // _M
_MAIN = _NBLK * _M
_TAIL = _B0 - _MAIN
_TB = 64
_P = _NBLK * _RBO + _TAIL
_B0V = _P * _PACK


def _compact_block(i0, i1, i2, i3, o_ref):
    o_ref[...] = jnp.concatenate(
        [i0[...], i1[...], i2[...], i3[...]], axis=0
    ).T


def _tail_block(i_ref, full_ref, o_ref):
    del full_ref
    o_ref[...] = jnp.concatenate([i_ref[...]] * _PACK, axis=1)


def _compact_table(wt, weight):
    in_specs = [
        pl.BlockSpec(
            (_D, _RBO), functools.partial(lambda k, i: (0, 4 * i + k), k)
        )
        for k in range(_PACK)
    ]
    main = pl.pallas_call(
        _compact_block,
        grid=(_NBLK,),
        in_specs=in_specs,
        out_specs=pl.BlockSpec((_RBO, _PACK * _D), lambda i: (i, 0)),
        out_shape=jax.ShapeDtypeStruct((_P, _PACK * _D), jnp.float32),
        compiler_params=pltpu.CompilerParams(
            dimension_semantics=("parallel",)
        ),
    )(wt, wt, wt, wt)
    tail = jax.lax.slice(weight, (_MAIN, 0), (_B0, _D))
    return pl.pallas_call(
        _tail_block,
        grid=(_TAIL // _TB,),
        in_specs=[
            pl.BlockSpec((_TB, _D), lambda j: (j, 0)),
            pl.BlockSpec(memory_space=pl.ANY),
        ],
        out_specs=pl.BlockSpec((_TB, _PACK * _D), lambda j: (_NBLK * _RBO // _TB + j, 0)),
        out_shape=jax.ShapeDtypeStruct((_P, _PACK * _D), jnp.float32),
        input_output_aliases={1: 0},
    )(tail, main)


def kernel(x, weight):
    weight = _compact_table(weight.T, weight).reshape(_B0V, _D)
    idx = x.T.reshape(_B)
    t = idx % _M
    idx = jnp.where(
        idx < _MAIN,
        (idx - t) + _PACK * (t % _RBO) + t // _RBO,
        _PACK * idx - (_PACK - 1) * _MAIN,
    )
    mesh = plsc.VectorSubcoreMesh(core_axis_name="c", subcore_axis_name="s")

    @functools.partial(
        pl.kernel,
        mesh=mesh,
        out_type=jax.ShapeDtypeStruct((_B, _D), jnp.float32),
        compiler_params=pltpu.CompilerParams(use_tc_tiling_on_sc=False),
        scratch_types=[
            pltpu.VMEM((_PER_W,), jnp.int32),
            pltpu.VMEM((_C, _D), jnp.float32),
            pltpu.VMEM((_C, _D), jnp.float32),
            pltpu.SemaphoreType.DMA,
            pltpu.SemaphoreType.DMA,
        ],
    )
    def gather_kernel(w_hbm, i_hbm, o_hbm, idx_v, rows0, rows1, sem0, sem1):
        wid = lax.axis_index("s") * _NC + lax.axis_index("c")
        base = wid * _PER_W

        pltpu.sync_copy(i_hbm.at[pl.ds(base, _PER_W)], idx_v)

        def start(j, rows, sem):
            pltpu.async_copy(w_hbm.at[idx_v.at[pl.ds(j * _C, _C)]], rows, sem)

        def wait(rows, sem):
            pltpu.make_async_copy(w_hbm.at[pl.ds(0, _C)], rows, sem).wait()

        def store(j, rows):
            pltpu.sync_copy(rows, o_hbm.at[pl.ds(base + j * _C, _C)])

        start(0, rows0, sem0)
        start(1, rows1, sem1)

        @pl.loop(0, _NCHUNK - 2, step=2)
        def _(j):
            wait(rows0, sem0)
            store(j, rows0)
            start(j + 2, rows0, sem0)
            wait(rows1, sem1)
            store(j + 1, rows1)
            start(j + 3, rows1, sem1)

        wait(rows0, sem0)
        store(_NCHUNK - 2, rows0)
        wait(rows1, sem1)
        store(_NCHUNK - 1, rows1)

    out = gather_kernel(weight, idx)
    h = jnp.swapaxes(out.reshape(26, 16384, _D), 1, 2)
    return h.transpose(2, 0, 1)
